# Initial kernel scaffold; baseline (speedup 1.0000x reference)
import functools

import jax
import jax.numpy as jnp
from jax import lax
from jax.experimental import pallas as pl
from jax.experimental.pallas import tpu as pltpu
from jax.experimental.pallas import tpu_sc as plsc

NC = 2
NS = 16
LANES = 16
NW = NC * NS

KA = 80
KB = 80


def _mesh():
    return plsc.VectorSubcoreMesh(
        core_axis_name="c", subcore_axis_name="s", num_cores=NC, num_subcores=NS
    )


def _sc_degree(dst3, n_nodes):
    chunks = dst3.shape[1]

    def body(dst_hbm, out_hbm, dst_v, deg_v):
        c = lax.axis_index("c")
        s = lax.axis_index("s")
        w = s * NC + c
        pltpu.sync_copy(dst_hbm.at[w], dst_v)

        def zero(i, carry):
            deg_v[pl.ds(i * LANES, LANES)] = jnp.zeros((LANES,), jnp.float32)
            return carry

        lax.fori_loop(0, n_nodes // LANES, zero, 0)

        ones = jnp.ones((LANES,), jnp.float32)

        def count(i, carry):
            idx = dst_v[i]
            plsc.addupdate_scatter(deg_v, [idx], ones)
            return carry

        lax.fori_loop(0, chunks, count, 0)
        pltpu.sync_copy(deg_v, out_hbm.at[w])

    kern = pl.kernel(
        body,
        out_type=jax.ShapeDtypeStruct((NW, n_nodes), jnp.float32),
        mesh=_mesh(),
        scratch_types=[
            pltpu.VMEM((chunks, LANES), jnp.int32),
            pltpu.VMEM((n_nodes,), jnp.float32),
        ],
    )
    return kern(dst3)


def _sc_pass_a(zs1a, zs1b, src3, dst3):
    n_nodes, d = zs1a.shape
    ca = src3.shape[1]
    rpt = n_nodes // NS

    def body(za_hbm, zb_hbm, src_hbm, dst_hbm, out_hbm, src_v, dst_v, rows_v, acc_sh):
        c = lax.axis_index("c")
        s = lax.axis_index("s")
        rs = pl.ds(s * rpt, rpt)

        @pl.when(c == 0)
        def _():
            pltpu.sync_copy(za_hbm.at[rs], acc_sh.at[rs])

        @pl.when(c == 1)
        def _():
            pltpu.sync_copy(zb_hbm.at[rs], acc_sh.at[rs])

        pltpu.sync_copy(src_hbm.at[s], src_v)
        pltpu.sync_copy(dst_hbm.at[s], dst_v)
        plsc.subcore_barrier()

        def chunk(j, carry):
            @pl.when(c == 0)
            def _():
                pltpu.sync_copy(za_hbm.at[src_v.at[j]], rows_v)

            @pl.when(c == 1)
            def _():
                pltpu.sync_copy(zb_hbm.at[src_v.at[j]], rows_v)

            pltpu.sync_copy(rows_v, acc_sh.at[dst_v.at[j]], add=True)
            return carry

        lax.fori_loop(0, ca, chunk, 0)
        plsc.subcore_barrier()
        pltpu.sync_copy(acc_sh.at[rs], out_hbm.at[c].at[rs])

    kern = pl.kernel(
        body,
        out_type=jax.ShapeDtypeStruct((NC, n_nodes, d), jnp.float32),
        mesh=_mesh(),
        scratch_types=[
            pltpu.VMEM((ca, KA), jnp.int32),
            pltpu.VMEM((ca, KA), jnp.int32),
            pltpu.VMEM((KA, d), jnp.float32),
            pltpu.VMEM_SHARED((n_nodes, d), jnp.float32),
        ],
    )
    return kern(zs1a, zs1b, src3, dst3)


def _sc_pass_b(zs2, src3, dst3):
    n_nodes, d = zs2.shape
    cb = src3.shape[1]
    rpt = n_nodes // NS

    def body(z_hbm, src_hbm, dst_hbm, out_hbm, src_v, dst_v, rows_v, acc_sh):
        c = lax.axis_index("c")
        s = lax.axis_index("s")
        w = s * NC + c
        rs = pl.ds(s * rpt, rpt)
        pltpu.sync_copy(z_hbm.at[rs], acc_sh.at[rs])
        pltpu.sync_copy(src_hbm.at[w], src_v)
        pltpu.sync_copy(dst_hbm.at[w], dst_v)
        plsc.subcore_barrier()

        def chunk(j, carry):
            pltpu.sync_copy(z_hbm.at[src_v.at[j]], rows_v)
            pltpu.sync_copy(rows_v, acc_sh.at[dst_v.at[j]], add=True)
            return carry

        lax.fori_loop(0, cb, chunk, 0)
        plsc.subcore_barrier()
        pltpu.sync_copy(acc_sh.at[rs], out_hbm.at[c].at[rs])

    kern = pl.kernel(
        body,
        out_type=jax.ShapeDtypeStruct((NC, n_nodes, d), jnp.float32),
        mesh=_mesh(),
        scratch_types=[
            pltpu.VMEM((cb, KB), jnp.int32),
            pltpu.VMEM((cb, KB), jnp.int32),
            pltpu.VMEM((KB, d), jnp.float32),
            pltpu.VMEM_SHARED((n_nodes, d), jnp.float32),
        ],
    )
    return kern(zs2, src3, dst3)


_R = 1000


def _prep_body(degt_ref, x_ref, h_ref, dinvb_ref, zs1a_ref, zs1b_ref):
    deg = jnp.sum(degt_ref[...], axis=1, keepdims=True) + 1.0
    dinv = lax.rsqrt(deg)
    dinvb_ref[...] = jnp.broadcast_to(dinv, x_ref.shape)
    zs1a_ref[...] = x_ref[...] * dinv
    zs1b_ref[...] = h_ref[...] * dinv


def _tc_prep(degt, x, h):
    n, d = x.shape
    grid = (n // _R,)
    row = pl.BlockSpec((_R, d), lambda i: (i, 0))
    return pl.pallas_call(
        _prep_body,
        grid=grid,
        in_specs=[pl.BlockSpec((_R, NW), lambda i: (i, 0)), row, row],
        out_specs=[row, row, row],
        out_shape=[jax.ShapeDtypeStruct((n, d), jnp.float32)] * 3,
    )(degt, x, h)


def _mm1_body(s1_ref, dinvb_ref, w1_ref, b1_ref, ru_ref, aggf_ref):
    dinv = dinvb_ref[...]
    a0 = s1_ref[0] * dinv
    a1 = s1_ref[1] * dinv
    y = (
        jnp.dot(a0, w1_ref[0], preferred_element_type=jnp.float32)
        + jnp.dot(a1, w1_ref[1], preferred_element_type=jnp.float32)
        + b1_ref[...]
    )
    ru_ref[...] = jax.nn.sigmoid(y)
    aggf_ref[...] = a0


def _tc_mm1(s1, dinvb, w1, b1):
    n, d = dinvb.shape
    do = w1.shape[2]
    grid = (n // _R,)
    return pl.pallas_call(
        _mm1_body,
        grid=grid,
        in_specs=[
            pl.BlockSpec((NC, _R, d), lambda i: (0, i, 0)),
            pl.BlockSpec((_R, d), lambda i: (i, 0)),
            pl.BlockSpec(w1.shape, lambda i: (0, 0, 0)),
            pl.BlockSpec((1, do), lambda i: (0, 0)),
        ],
        out_specs=[
            pl.BlockSpec((_R, do), lambda i: (i, 0)),
            pl.BlockSpec((_R, d), lambda i: (i, 0)),
        ],
        out_shape=[
            jax.ShapeDtypeStruct((n, do), jnp.float32),
            jax.ShapeDtypeStruct((n, d), jnp.float32),
        ],
    )(s1, dinvb, w1, b1)


def _ew2_body(r_ref, h_ref, dinvb_ref, zs2_ref):
    zs2_ref[...] = r_ref[...] * h_ref[...] * dinvb_ref[...]


def _tc_ew2(r, h, dinvb):
    n, d = r.shape
    row = pl.BlockSpec((_R, d), lambda i: (i, 0))
    return pl.pallas_call(
        _ew2_body,
        grid=(n // _R,),
        in_specs=[row, row, row],
        out_specs=row,
        out_shape=jax.ShapeDtypeStruct((n, d), jnp.float32),
    )(r, h, dinvb)


def _mm2_body(s2_ref, zs2_ref, dinvb_ref, aggf_ref, u_ref, h_ref, w2_ref, b2_ref, out_ref):
    agg_b = (s2_ref[0] + s2_ref[1] - zs2_ref[...]) * dinvb_ref[...]
    y = (
        jnp.dot(aggf_ref[...], w2_ref[0], preferred_element_type=jnp.float32)
        + jnp.dot(agg_b, w2_ref[1], preferred_element_type=jnp.float32)
        + b2_ref[...]
    )
    cand = jnp.tanh(y)
    u = u_ref[...]
    out_ref[...] = u * h_ref[...] + (1.0 - u) * cand


def _tc_mm2(s2, zs2, dinvb, aggf, u, h, w2, b2):
    n, d = zs2.shape
    do = w2.shape[2]
    grid = (n // _R,)
    row = pl.BlockSpec((_R, d), lambda i: (i, 0))
    return pl.pallas_call(
        _mm2_body,
        grid=grid,
        in_specs=[
            pl.BlockSpec((NC, _R, d), lambda i: (0, i, 0)),
            row,
            row,
            row,
            row,
            row,
            pl.BlockSpec(w2.shape, lambda i: (0, 0, 0)),
            pl.BlockSpec((1, do), lambda i: (0, 0)),
        ],
        out_specs=pl.BlockSpec((_R, do), lambda i: (i, 0)),
        out_shape=jax.ShapeDtypeStruct((n, do), jnp.float32),
    )(s2, zs2, dinvb, aggf, u, h, w2, b2)


def kernel(x, edge_index, hidden_state, W1, b1, W2, b2):
    n, d = x.shape
    nh = hidden_state.shape[1]
    e = edge_index.shape[1]

    src = edge_index[0].astype(jnp.int32)
    dst = edge_index[1].astype(jnp.int32)
    dst3_16 = dst.reshape(NW, e // NW // LANES, LANES)
    src_a = src.reshape(NS, e // NS // KA, KA)
    dst_a = dst.reshape(NS, e // NS // KA, KA)
    src_b = src.reshape(NW, e // NW // KB, KB)
    dst_b = dst.reshape(NW, e // NW // KB, KB)

    degp = _sc_degree(dst3_16, n)
    dinvb, zs1a, zs1b = _tc_prep(degp.T, x, hidden_state)
    s1 = _sc_pass_a(zs1a, zs1b, src_a, dst_a)
    ru, aggf = _tc_mm1(
        s1, dinvb, W1.reshape(2, d, 2 * nh), b1.reshape(1, 2 * nh)
    )
    r = ru[: n // 2].reshape(n, nh)
    u = ru[n // 2 :].reshape(n, nh)
    zs2 = _tc_ew2(r, hidden_state, dinvb)
    s2 = _sc_pass_b(zs2, src_b, dst_b)
    return _tc_mm2(
        s2, zs2, dinvb, aggf, u, hidden_state, W2.reshape(2, d, nh), b2.reshape(1, nh)
    )


# speedup vs baseline: 14.5206x; 14.5206x over previous
import functools

import jax
import jax.numpy as jnp
from jax import lax
from jax.experimental import pallas as pl
from jax.experimental.pallas import tpu as pltpu
from jax.experimental.pallas import tpu_sc as plsc

NC = 2
NS = 16
LANES = 16
NW = NC * NS

KA = 80
KB = 80


def _mesh():
    return plsc.VectorSubcoreMesh(
        core_axis_name="c", subcore_axis_name="s", num_cores=NC, num_subcores=NS
    )


_SC_PARAMS = pltpu.CompilerParams(
    needs_layout_passes=False, use_tc_tiling_on_sc=False
)


def _sc_degree(dst3, n_nodes):
    chunks = dst3.shape[1]

    def body(dst_hbm, out_hbm, dst_v, deg_v):
        c = lax.axis_index("c")
        s = lax.axis_index("s")
        w = s * NC + c
        pltpu.sync_copy(dst_hbm.at[w], dst_v)

        def zero(i, carry):
            deg_v[pl.ds(i * LANES, LANES)] = jnp.zeros((LANES,), jnp.float32)
            return carry

        lax.fori_loop(0, n_nodes // LANES, zero, 0)

        ones = jnp.ones((LANES,), jnp.float32)

        def count(i, carry):
            idx = dst_v[i]
            plsc.addupdate_scatter(deg_v, [idx], ones)
            return carry

        lax.fori_loop(0, chunks, count, 0)
        pltpu.sync_copy(deg_v, out_hbm.at[w])

    kern = pl.kernel(
        body,
        out_type=jax.ShapeDtypeStruct((NW, n_nodes), jnp.float32),
        mesh=_mesh(),
        compiler_params=_SC_PARAMS,
        scratch_types=[
            pltpu.VMEM((chunks, LANES), jnp.int32),
            pltpu.VMEM((n_nodes,), jnp.float32),
        ],
    )
    return kern(dst3)


def _sc_pass_a(zs1a, zs1b, src3, dst3):
    n_nodes, d = zs1a.shape
    ca = src3.shape[1]
    rpt = n_nodes // NS

    def body(za_hbm, zb_hbm, src_hbm, dst_hbm, out_hbm, src_v, dst_v, rows_v, acc_sh):
        c = lax.axis_index("c")
        s = lax.axis_index("s")
        rs = pl.ds(s * rpt, rpt)

        @pl.when(c == 0)
        def _():
            pltpu.sync_copy(za_hbm.at[rs], acc_sh.at[rs])

        @pl.when(c == 1)
        def _():
            pltpu.sync_copy(zb_hbm.at[rs], acc_sh.at[rs])

        pltpu.sync_copy(src_hbm.at[s], src_v)
        pltpu.sync_copy(dst_hbm.at[s], dst_v)
        plsc.subcore_barrier()

        def chunk(j, carry):
            @pl.when(c == 0)
            def _():
                pltpu.sync_copy(za_hbm.at[src_v.at[j]], rows_v)

            @pl.when(c == 1)
            def _():
                pltpu.sync_copy(zb_hbm.at[src_v.at[j]], rows_v)

            pltpu.sync_copy(rows_v, acc_sh.at[dst_v.at[j]], add=True)
            return carry

        lax.fori_loop(0, ca, chunk, 0)
        plsc.subcore_barrier()
        pltpu.sync_copy(acc_sh.at[rs], out_hbm.at[c].at[rs])

    kern = pl.kernel(
        body,
        out_type=jax.ShapeDtypeStruct((NC, n_nodes, d), jnp.float32),
        mesh=_mesh(),
        compiler_params=_SC_PARAMS,
        scratch_types=[
            pltpu.VMEM((ca, KA), jnp.int32),
            pltpu.VMEM((ca, KA), jnp.int32),
            pltpu.VMEM((KA, d), jnp.float32),
            pltpu.VMEM_SHARED((n_nodes, d), jnp.float32),
        ],
    )
    return kern(zs1a, zs1b, src3, dst3)


def _sc_pass_b(zs2, src3, dst3):
    n_nodes, d = zs2.shape
    cb = src3.shape[1]
    rpt = n_nodes // NS

    def body(z_hbm, src_hbm, dst_hbm, out_hbm, src_v, dst_v, rows_v, acc_sh):
        c = lax.axis_index("c")
        s = lax.axis_index("s")
        w = s * NC + c
        rs = pl.ds(s * rpt, rpt)
        pltpu.sync_copy(z_hbm.at[rs], acc_sh.at[rs])
        pltpu.sync_copy(src_hbm.at[w], src_v)
        pltpu.sync_copy(dst_hbm.at[w], dst_v)
        plsc.subcore_barrier()

        def chunk(j, carry):
            pltpu.sync_copy(z_hbm.at[src_v.at[j]], rows_v)
            pltpu.sync_copy(rows_v, acc_sh.at[dst_v.at[j]], add=True)
            return carry

        lax.fori_loop(0, cb, chunk, 0)
        plsc.subcore_barrier()
        pltpu.sync_copy(acc_sh.at[rs], out_hbm.at[c].at[rs])

    kern = pl.kernel(
        body,
        out_type=jax.ShapeDtypeStruct((NC, n_nodes, d), jnp.float32),
        mesh=_mesh(),
        compiler_params=_SC_PARAMS,
        scratch_types=[
            pltpu.VMEM((cb, KB), jnp.int32),
            pltpu.VMEM((cb, KB), jnp.int32),
            pltpu.VMEM((KB, d), jnp.float32),
            pltpu.VMEM_SHARED((n_nodes, d), jnp.float32),
        ],
    )
    return kern(zs2, src3, dst3)


_R = 1000


def _prep_body(degt_ref, x_ref, h_ref, dinvb_ref, zs1a_ref, zs1b_ref):
    deg = jnp.sum(degt_ref[...], axis=1, keepdims=True) + 1.0
    dinv = lax.rsqrt(deg)
    dinvb_ref[...] = jnp.broadcast_to(dinv, x_ref.shape)
    zs1a_ref[...] = x_ref[...] * dinv
    zs1b_ref[...] = h_ref[...] * dinv


def _tc_prep(degt, x, h):
    n, d = x.shape
    grid = (n // _R,)
    row = pl.BlockSpec((_R, d), lambda i: (i, 0))
    return pl.pallas_call(
        _prep_body,
        grid=grid,
        in_specs=[pl.BlockSpec((_R, NW), lambda i: (i, 0)), row, row],
        out_specs=[row, row, row],
        out_shape=[jax.ShapeDtypeStruct((n, d), jnp.float32)] * 3,
    )(degt, x, h)


def _mm1_body(s1_ref, dinvb_ref, w1_ref, b1_ref, ru_ref, aggf_ref):
    dinv = dinvb_ref[...]
    a0 = s1_ref[0] * dinv
    a1 = s1_ref[1] * dinv
    y = (
        jnp.dot(a0, w1_ref[0], preferred_element_type=jnp.float32)
        + jnp.dot(a1, w1_ref[1], preferred_element_type=jnp.float32)
        + b1_ref[...]
    )
    ru_ref[...] = jax.nn.sigmoid(y)
    aggf_ref[...] = a0


def _tc_mm1(s1, dinvb, w1, b1):
    n, d = dinvb.shape
    do = w1.shape[2]
    grid = (n // _R,)
    return pl.pallas_call(
        _mm1_body,
        grid=grid,
        in_specs=[
            pl.BlockSpec((NC, _R, d), lambda i: (0, i, 0)),
            pl.BlockSpec((_R, d), lambda i: (i, 0)),
            pl.BlockSpec(w1.shape, lambda i: (0, 0, 0)),
            pl.BlockSpec((1, do), lambda i: (0, 0)),
        ],
        out_specs=[
            pl.BlockSpec((_R, do), lambda i: (i, 0)),
            pl.BlockSpec((_R, d), lambda i: (i, 0)),
        ],
        out_shape=[
            jax.ShapeDtypeStruct((n, do), jnp.float32),
            jax.ShapeDtypeStruct((n, d), jnp.float32),
        ],
    )(s1, dinvb, w1, b1)


def _ew2_body(r_ref, h_ref, dinvb_ref, zs2_ref):
    zs2_ref[...] = r_ref[...] * h_ref[...] * dinvb_ref[...]


def _tc_ew2(r, h, dinvb):
    n, d = r.shape
    row = pl.BlockSpec((_R, d), lambda i: (i, 0))
    return pl.pallas_call(
        _ew2_body,
        grid=(n // _R,),
        in_specs=[row, row, row],
        out_specs=row,
        out_shape=jax.ShapeDtypeStruct((n, d), jnp.float32),
    )(r, h, dinvb)


def _mm2_body(s2_ref, zs2_ref, dinvb_ref, aggf_ref, u_ref, h_ref, w2_ref, b2_ref, out_ref):
    agg_b = (s2_ref[0] + s2_ref[1] - zs2_ref[...]) * dinvb_ref[...]
    y = (
        jnp.dot(aggf_ref[...], w2_ref[0], preferred_element_type=jnp.float32)
        + jnp.dot(agg_b, w2_ref[1], preferred_element_type=jnp.float32)
        + b2_ref[...]
    )
    cand = jnp.tanh(y)
    u = u_ref[...]
    out_ref[...] = u * h_ref[...] + (1.0 - u) * cand


def _tc_mm2(s2, zs2, dinvb, aggf, u, h, w2, b2):
    n, d = zs2.shape
    do = w2.shape[2]
    grid = (n // _R,)
    row = pl.BlockSpec((_R, d), lambda i: (i, 0))
    return pl.pallas_call(
        _mm2_body,
        grid=grid,
        in_specs=[
            pl.BlockSpec((NC, _R, d), lambda i: (0, i, 0)),
            row,
            row,
            row,
            row,
            row,
            pl.BlockSpec(w2.shape, lambda i: (0, 0, 0)),
            pl.BlockSpec((1, do), lambda i: (0, 0)),
        ],
        out_specs=pl.BlockSpec((_R, do), lambda i: (i, 0)),
        out_shape=jax.ShapeDtypeStruct((n, do), jnp.float32),
    )(s2, zs2, dinvb, aggf, u, h, w2, b2)


def kernel(x, edge_index, hidden_state, W1, b1, W2, b2):
    n, d = x.shape
    nh = hidden_state.shape[1]
    e = edge_index.shape[1]

    src = edge_index[0].astype(jnp.int32)
    dst = edge_index[1].astype(jnp.int32)
    dst3_16 = dst.reshape(NW, e // NW // LANES, LANES)
    src_a = src.reshape(NS, e // NS // KA, KA)
    dst_a = dst.reshape(NS, e // NS // KA, KA)
    src_b = src.reshape(NW, e // NW // KB, KB)
    dst_b = dst.reshape(NW, e // NW // KB, KB)

    degp = _sc_degree(dst3_16, n)
    dinvb, zs1a, zs1b = _tc_prep(degp.T, x, hidden_state)
    s1 = _sc_pass_a(zs1a, zs1b, src_a, dst_a)
    ru, aggf = _tc_mm1(
        s1, dinvb, W1.reshape(2, d, 2 * nh), b1.reshape(1, 2 * nh)
    )
    r = ru[: n // 2].reshape(n, nh)
    u = ru[n // 2 :].reshape(n, nh)
    zs2 = _tc_ew2(r, hidden_state, dinvb)
    s2 = _sc_pass_b(zs2, src_b, dst_b)
    return _tc_mm2(
        s2, zs2, dinvb, aggf, u, hidden_state, W2.reshape(2, d, nh), b2.reshape(1, nh)
    )


# speedup vs baseline: 25.8135x; 1.7777x over previous
import functools

import jax
import jax.numpy as jnp
from jax import lax
from jax.experimental import pallas as pl
from jax.experimental.pallas import tpu as pltpu
from jax.experimental.pallas import tpu_sc as plsc

NC = 2
NS = 16
LANES = 16
NW = NC * NS

KA = 80
KB = 80
NBUF = 4
LOOK = 2
GRP = 25


def _mesh():
    return plsc.VectorSubcoreMesh(
        core_axis_name="c", subcore_axis_name="s", num_cores=NC, num_subcores=NS
    )


_SC_PARAMS = pltpu.CompilerParams(
    needs_layout_passes=False, use_tc_tiling_on_sc=False
)


def _sc_degree(dst3, n_nodes):
    chunks = dst3.shape[1]

    def body(dst_hbm, out_hbm, dst_v, deg_v):
        c = lax.axis_index("c")
        s = lax.axis_index("s")
        w = s * NC + c
        pltpu.sync_copy(dst_hbm.at[w], dst_v)

        def zero(i, carry):
            deg_v[pl.ds(i * LANES, LANES)] = jnp.zeros((LANES,), jnp.float32)
            return carry

        lax.fori_loop(0, n_nodes // LANES, zero, 0)

        ones = jnp.ones((LANES,), jnp.float32)

        def count(i, carry):
            idx = dst_v[i]
            plsc.addupdate_scatter(deg_v, [idx], ones)
            return carry

        lax.fori_loop(0, chunks, count, 0)
        pltpu.sync_copy(deg_v, out_hbm.at[w])

    kern = pl.kernel(
        body,
        out_type=jax.ShapeDtypeStruct((NW, n_nodes), jnp.float32),
        mesh=_mesh(),
        compiler_params=_SC_PARAMS,
        scratch_types=[
            pltpu.VMEM((chunks, LANES), jnp.int32),
            pltpu.VMEM((n_nodes,), jnp.float32),
        ],
    )
    return kern(dst3)


def _sc_pass_a(zs1a, zs1b, src3, dst3):
    n_nodes, d = zs1a.shape
    ca = src3.shape[1]
    rpt = n_nodes // NS

    j_outer = (ca + 2 * NBUF - 1) // NBUF

    def body(za_hbm, zb_hbm, src_hbm, dst_hbm, out_hbm, src_i, dst_i, rows_v,
             acc_sh, sem_g, sem_s, sem_is, sem_id):
        c = lax.axis_index("c")
        s = lax.axis_index("s")
        rs = pl.ds(s * rpt, rpt)

        @pl.when(c == 0)
        def _():
            pltpu.sync_copy(za_hbm.at[rs], acc_sh.at[rs])

        @pl.when(c == 1)
        def _():
            pltpu.sync_copy(zb_hbm.at[rs], acc_sh.at[rs])

        pltpu.sync_copy(src_hbm.at[s, pl.ds(0, GRP)], src_i.at[0])
        pltpu.sync_copy(dst_hbm.at[s, pl.ds(0, GRP)], dst_i.at[0])
        plsc.subcore_barrier()

        def outer(jo, carry):
            jbase = jo * NBUF
            for bi in range(NBUF):
                j = jbase + bi
                bg = (bi - LOOK) % NBUF
                js = j - LOOK

                @pl.when(jnp.logical_and(j >= NBUF, j - NBUF < ca))
                def _():
                    pltpu.make_async_copy(
                        za_hbm.at[pl.ds(0, KA)], rows_v.at[bi], sem_s.at[bi]
                    ).wait()

                @pl.when(jnp.logical_and(j % GRP == NBUF, j - NBUF + GRP < ca))
                def _():
                    nxt = j - (j % GRP) + GRP
                    gb = (nxt // GRP) % 2
                    pltpu.async_copy(
                        src_hbm.at[s, pl.ds(nxt, GRP)], src_i.at[gb], sem_is
                    )
                    pltpu.async_copy(
                        dst_hbm.at[s, pl.ds(nxt, GRP)], dst_i.at[gb], sem_id
                    )

                @pl.when(jnp.logical_and(jnp.logical_and(j % GRP == 0, j > 0), j < ca))
                def _():
                    pltpu.make_async_copy(
                        src_hbm.at[s, pl.ds(0, GRP)], src_i.at[0], sem_is
                    ).wait()
                    pltpu.make_async_copy(
                        dst_hbm.at[s, pl.ds(0, GRP)], dst_i.at[0], sem_id
                    ).wait()

                @pl.when(jnp.logical_and(j < ca, c == 0))
                def _():
                    pltpu.async_copy(
                        za_hbm.at[src_i.at[(j // GRP) % 2, j % GRP]],
                        rows_v.at[bi], sem_g.at[bi],
                    )

                @pl.when(jnp.logical_and(j < ca, c == 1))
                def _():
                    pltpu.async_copy(
                        zb_hbm.at[src_i.at[(j // GRP) % 2, j % GRP]],
                        rows_v.at[bi], sem_g.at[bi],
                    )

                @pl.when(jnp.logical_and(js >= 0, js < ca))
                def _():
                    pltpu.make_async_copy(
                        za_hbm.at[pl.ds(0, KA)], rows_v.at[bg], sem_g.at[bg]
                    ).wait()
                    pltpu.async_copy(
                        rows_v.at[bg],
                        acc_sh.at[dst_i.at[(js // GRP) % 2, js % GRP]],
                        sem_s.at[bg], add=True,
                    )
            return carry

        lax.fori_loop(0, j_outer, outer, 0)
        plsc.subcore_barrier()
        pltpu.sync_copy(acc_sh.at[rs], out_hbm.at[c].at[rs])

    kern = pl.kernel(
        body,
        out_type=jax.ShapeDtypeStruct((NC, n_nodes, d), jnp.float32),
        mesh=_mesh(),
        compiler_params=_SC_PARAMS,
        scratch_types=[
            pltpu.VMEM((2, GRP, KA), jnp.int32),
            pltpu.VMEM((2, GRP, KA), jnp.int32),
            pltpu.VMEM((NBUF, KA, d), jnp.float32),
            pltpu.VMEM_SHARED((n_nodes, d), jnp.float32),
            pltpu.SemaphoreType.DMA((NBUF,)),
            pltpu.SemaphoreType.DMA((NBUF,)),
            pltpu.SemaphoreType.DMA,
            pltpu.SemaphoreType.DMA,
        ],
    )
    return kern(zs1a, zs1b, src3, dst3)


def _sc_pass_b(zs2, src3, dst3):
    n_nodes, d = zs2.shape
    cb = src3.shape[1]
    rpt = n_nodes // NS

    j_outer = (cb + 2 * NBUF - 1) // NBUF

    def body(z_hbm, src_hbm, dst_hbm, out_hbm, src_i, dst_i, rows_v, acc_sh,
             sem_g, sem_s, sem_is, sem_id):
        c = lax.axis_index("c")
        s = lax.axis_index("s")
        w = s * NC + c
        rs = pl.ds(s * rpt, rpt)
        pltpu.sync_copy(z_hbm.at[rs], acc_sh.at[rs])
        pltpu.sync_copy(src_hbm.at[w, pl.ds(0, GRP)], src_i.at[0])
        pltpu.sync_copy(dst_hbm.at[w, pl.ds(0, GRP)], dst_i.at[0])
        plsc.subcore_barrier()

        def outer(jo, carry):
            jbase = jo * NBUF
            for bi in range(NBUF):
                j = jbase + bi
                bg = (bi - LOOK) % NBUF
                js = j - LOOK

                @pl.when(jnp.logical_and(j >= NBUF, j - NBUF < cb))
                def _():
                    pltpu.make_async_copy(
                        z_hbm.at[pl.ds(0, KB)], rows_v.at[bi], sem_s.at[bi]
                    ).wait()

                @pl.when(jnp.logical_and(j % GRP == NBUF, j - NBUF + GRP < cb))
                def _():
                    nxt = j - (j % GRP) + GRP
                    gb = (nxt // GRP) % 2
                    pltpu.async_copy(
                        src_hbm.at[w, pl.ds(nxt, GRP)], src_i.at[gb], sem_is
                    )
                    pltpu.async_copy(
                        dst_hbm.at[w, pl.ds(nxt, GRP)], dst_i.at[gb], sem_id
                    )

                @pl.when(jnp.logical_and(jnp.logical_and(j % GRP == 0, j > 0), j < cb))
                def _():
                    pltpu.make_async_copy(
                        src_hbm.at[w, pl.ds(0, GRP)], src_i.at[0], sem_is
                    ).wait()
                    pltpu.make_async_copy(
                        dst_hbm.at[w, pl.ds(0, GRP)], dst_i.at[0], sem_id
                    ).wait()

                @pl.when(j < cb)
                def _():
                    pltpu.async_copy(
                        z_hbm.at[src_i.at[(j // GRP) % 2, j % GRP]],
                        rows_v.at[bi], sem_g.at[bi],
                    )

                @pl.when(jnp.logical_and(js >= 0, js < cb))
                def _():
                    pltpu.make_async_copy(
                        z_hbm.at[pl.ds(0, KB)], rows_v.at[bg], sem_g.at[bg]
                    ).wait()
                    pltpu.async_copy(
                        rows_v.at[bg],
                        acc_sh.at[dst_i.at[(js // GRP) % 2, js % GRP]],
                        sem_s.at[bg], add=True,
                    )
            return carry

        lax.fori_loop(0, j_outer, outer, 0)
        plsc.subcore_barrier()
        pltpu.sync_copy(acc_sh.at[rs], out_hbm.at[c].at[rs])

    kern = pl.kernel(
        body,
        out_type=jax.ShapeDtypeStruct((NC, n_nodes, d), jnp.float32),
        mesh=_mesh(),
        compiler_params=_SC_PARAMS,
        scratch_types=[
            pltpu.VMEM((2, GRP, KB), jnp.int32),
            pltpu.VMEM((2, GRP, KB), jnp.int32),
            pltpu.VMEM((NBUF, KB, d), jnp.float32),
            pltpu.VMEM_SHARED((n_nodes, d), jnp.float32),
            pltpu.SemaphoreType.DMA((NBUF,)),
            pltpu.SemaphoreType.DMA((NBUF,)),
            pltpu.SemaphoreType.DMA,
            pltpu.SemaphoreType.DMA,
        ],
    )
    return kern(zs2, src3, dst3)


_R = 1000


def _prep_body(degt_ref, x_ref, h_ref, dinvb_ref, zs1a_ref, zs1b_ref):
    deg = jnp.sum(degt_ref[...], axis=1, keepdims=True) + 1.0
    dinv = lax.rsqrt(deg)
    dinvb_ref[...] = jnp.broadcast_to(dinv, x_ref.shape)
    zs1a_ref[...] = x_ref[...] * dinv
    zs1b_ref[...] = h_ref[...] * dinv


def _tc_prep(degt, x, h):
    n, d = x.shape
    grid = (n // _R,)
    row = pl.BlockSpec((_R, d), lambda i: (i, 0))
    return pl.pallas_call(
        _prep_body,
        grid=grid,
        in_specs=[pl.BlockSpec((_R, NW), lambda i: (i, 0)), row, row],
        out_specs=[row, row, row],
        out_shape=[jax.ShapeDtypeStruct((n, d), jnp.float32)] * 3,
    )(degt, x, h)


def _mm1_body(s1_ref, dinvb_ref, w1_ref, b1_ref, ru_ref, aggf_ref):
    dinv = dinvb_ref[...]
    a0 = s1_ref[0] * dinv
    a1 = s1_ref[1] * dinv
    y = (
        jnp.dot(a0, w1_ref[0], preferred_element_type=jnp.float32)
        + jnp.dot(a1, w1_ref[1], preferred_element_type=jnp.float32)
        + b1_ref[...]
    )
    ru_ref[...] = jax.nn.sigmoid(y)
    aggf_ref[...] = a0


def _tc_mm1(s1, dinvb, w1, b1):
    n, d = dinvb.shape
    do = w1.shape[2]
    grid = (n // _R,)
    return pl.pallas_call(
        _mm1_body,
        grid=grid,
        in_specs=[
            pl.BlockSpec((NC, _R, d), lambda i: (0, i, 0)),
            pl.BlockSpec((_R, d), lambda i: (i, 0)),
            pl.BlockSpec(w1.shape, lambda i: (0, 0, 0)),
            pl.BlockSpec((1, do), lambda i: (0, 0)),
        ],
        out_specs=[
            pl.BlockSpec((_R, do), lambda i: (i, 0)),
            pl.BlockSpec((_R, d), lambda i: (i, 0)),
        ],
        out_shape=[
            jax.ShapeDtypeStruct((n, do), jnp.float32),
            jax.ShapeDtypeStruct((n, d), jnp.float32),
        ],
    )(s1, dinvb, w1, b1)


def _ew2_body(r_ref, h_ref, dinvb_ref, zs2_ref):
    zs2_ref[...] = r_ref[...] * h_ref[...] * dinvb_ref[...]


def _tc_ew2(r, h, dinvb):
    n, d = r.shape
    row = pl.BlockSpec((_R, d), lambda i: (i, 0))
    return pl.pallas_call(
        _ew2_body,
        grid=(n // _R,),
        in_specs=[row, row, row],
        out_specs=row,
        out_shape=jax.ShapeDtypeStruct((n, d), jnp.float32),
    )(r, h, dinvb)


def _mm2_body(s2_ref, zs2_ref, dinvb_ref, aggf_ref, u_ref, h_ref, w2_ref, b2_ref, out_ref):
    agg_b = (s2_ref[0] + s2_ref[1] - zs2_ref[...]) * dinvb_ref[...]
    y = (
        jnp.dot(aggf_ref[...], w2_ref[0], preferred_element_type=jnp.float32)
        + jnp.dot(agg_b, w2_ref[1], preferred_element_type=jnp.float32)
        + b2_ref[...]
    )
    cand = jnp.tanh(y)
    u = u_ref[...]
    out_ref[...] = u * h_ref[...] + (1.0 - u) * cand


def _tc_mm2(s2, zs2, dinvb, aggf, u, h, w2, b2):
    n, d = zs2.shape
    do = w2.shape[2]
    grid = (n // _R,)
    row = pl.BlockSpec((_R, d), lambda i: (i, 0))
    return pl.pallas_call(
        _mm2_body,
        grid=grid,
        in_specs=[
            pl.BlockSpec((NC, _R, d), lambda i: (0, i, 0)),
            row,
            row,
            row,
            row,
            row,
            pl.BlockSpec(w2.shape, lambda i: (0, 0, 0)),
            pl.BlockSpec((1, do), lambda i: (0, 0)),
        ],
        out_specs=pl.BlockSpec((_R, do), lambda i: (i, 0)),
        out_shape=jax.ShapeDtypeStruct((n, do), jnp.float32),
    )(s2, zs2, dinvb, aggf, u, h, w2, b2)


def kernel(x, edge_index, hidden_state, W1, b1, W2, b2):
    n, d = x.shape
    nh = hidden_state.shape[1]
    e = edge_index.shape[1]

    src = edge_index[0].astype(jnp.int32)
    dst = edge_index[1].astype(jnp.int32)
    dst3_16 = dst.reshape(NW, e // NW // LANES, LANES)
    src_a = src.reshape(NS, e // NS // KA, KA)
    dst_a = dst.reshape(NS, e // NS // KA, KA)
    src_b = src.reshape(NW, e // NW // KB, KB)
    dst_b = dst.reshape(NW, e // NW // KB, KB)

    degp = _sc_degree(dst3_16, n)
    dinvb, zs1a, zs1b = _tc_prep(degp.T, x, hidden_state)
    s1 = _sc_pass_a(zs1a, zs1b, src_a, dst_a)
    ru, aggf = _tc_mm1(
        s1, dinvb, W1.reshape(2, d, 2 * nh), b1.reshape(1, 2 * nh)
    )
    r = ru[: n // 2].reshape(n, nh)
    u = ru[n // 2 :].reshape(n, nh)
    zs2 = _tc_ew2(r, hidden_state, dinvb)
    s2 = _sc_pass_b(zs2, src_b, dst_b)
    return _tc_mm2(
        s2, zs2, dinvb, aggf, u, hidden_state, W2.reshape(2, d, nh), b2.reshape(1, nh)
    )
